# baseline (speedup 1.0000x reference)
import functools

import jax
import jax.numpy as jnp
from jax import lax
from jax.experimental import pallas as pl
from jax.experimental.pallas import tpu as pltpu
from jax.experimental.pallas import tpu_sc as plsc

VOCAB = 1000000
EMBED = 32
NFEAT = 26
BATCH = 4096
HIDDEN = 512
NCLASS = 2

NC = 2
NS = 16
NW = NC * NS

NROWS = NFEAT * EMBED
RPW = NROWS // NW


def _gather_t(embT, x):
    mesh = plsc.VectorSubcoreMesh(core_axis_name="c", subcore_axis_name="s")

    @functools.partial(
        pl.kernel,
        out_type=jax.ShapeDtypeStruct((NROWS, BATCH), jnp.float32),
        mesh=mesh,
        scratch_types=[
            pltpu.VMEM((2, BATCH), jnp.int32),
            pltpu.VMEM((RPW, BATCH), jnp.float32),
            pltpu.SemaphoreType.DMA,
        ],
        compiler_params=pltpu.CompilerParams(use_tc_tiling_on_sc=False),
    )
    def gather_kernel(embT_hbm, x_hbm, out_hbm, xf_v, buf_v, gsem):
        wid = lax.axis_index("s") * NC + lax.axis_index("c")
        r0 = wid * RPW
        f0 = jnp.minimum(r0 // EMBED, NFEAT - 2)
        pltpu.sync_copy(x_hbm.at[pl.ds(f0, 2)], xf_v)

        def fire(rr, _):
            r = r0 + rr
            fi = r // EMBED - f0
            j = lax.rem(r, EMBED)
            pltpu.async_copy(
                embT_hbm.at[j].at[xf_v.at[fi]], buf_v.at[rr], gsem
            )
            return ()

        lax.fori_loop(0, RPW, fire, (), unroll=False)

        def drain(rr, _):
            r = r0 + rr
            fi = r // EMBED - f0
            j = lax.rem(r, EMBED)
            pltpu.make_async_copy(
                embT_hbm.at[j].at[xf_v.at[fi]], buf_v.at[rr], gsem
            ).wait()
            return ()

        lax.fori_loop(0, RPW, drain, (), unroll=False)

        pltpu.sync_copy(buf_v, out_hbm.at[pl.ds(r0, RPW)])

    return gather_kernel(embT, x)


def _mlp_block(x_ref, w1_ref, b1_ref, w2t_ref, b2_ref, o_ref):
    h = lax.dot_general(
        w1_ref[...], x_ref[...], (((0,), (0,)), ((), ())),
        preferred_element_type=jnp.float32,
    )
    h = jnp.maximum(h + b1_ref[...], 0.0)
    o_ref[...] = (
        jnp.dot(w2t_ref[...], h, preferred_element_type=jnp.float32)
        + b2_ref[...]
    )


def _mlp_t(eT, W1, b1, W2t, b2):
    bb = 512
    grid = (BATCH // bb,)
    return pl.pallas_call(
        _mlp_block,
        grid=grid,
        in_specs=[
            pl.BlockSpec((NROWS, bb), lambda i: (0, i)),
            pl.BlockSpec((NROWS, HIDDEN), lambda i: (0, 0)),
            pl.BlockSpec((HIDDEN, 1), lambda i: (0, 0)),
            pl.BlockSpec((NCLASS, HIDDEN), lambda i: (0, 0)),
            pl.BlockSpec((NCLASS, 1), lambda i: (0, 0)),
        ],
        out_specs=pl.BlockSpec((NCLASS, bb), lambda i: (0, i)),
        out_shape=jax.ShapeDtypeStruct((NCLASS, BATCH), jnp.float32),
    )(eT, W1, b1.reshape(HIDDEN, 1), W2t, b2.reshape(NCLASS, 1))


@jax.jit
def kernel(x, emb, W1, b1, W2, b2):
    embT = emb.T
    eT = _gather_t(embT, x)
    oT = _mlp_t(eT, W1, b1, W2.T, b2)
    return oT.T


# speedup vs baseline: 5.3208x; 5.3208x over previous
import functools

import jax
import jax.numpy as jnp
from jax import lax
from jax.experimental import pallas as pl
from jax.experimental.pallas import tpu as pltpu
from jax.experimental.pallas import tpu_sc as plsc

VOCAB = 1000000
EMBED = 32
NFEAT = 26
BATCH = 4096
HIDDEN = 512
NCLASS = 2

NC = 2
NS = 16
NW = NC * NS

BC = 2048
NB = (VOCAB + BC - 1) // BC
TROWS = NB * 512
DPAD = NFEAT * EMBED + 64


def _pack_table_block(x_ref, o_ref):
    t = jnp.transpose(x_ref[...], (1, 0))
    for s in range(4):
        o_ref[:, s * EMBED:(s + 1) * EMBED] = t[s * 512:(s + 1) * 512, :]


def _pack_table(embT):
    return pl.pallas_call(
        _pack_table_block,
        grid=(NB,),
        in_specs=[pl.BlockSpec((EMBED, BC), lambda i: (0, i))],
        out_specs=pl.BlockSpec((512, 128), lambda i: (i, 0)),
        out_shape=jax.ShapeDtypeStruct((TROWS, 128), jnp.float32),
    )(embT)


def _gather_rows(table, x):
    bpb = BATCH // NW
    bcn = 8
    bcs = bpb // bcn
    mesh = plsc.VectorSubcoreMesh(core_axis_name="c", subcore_axis_name="s")

    @functools.partial(
        pl.kernel,
        out_type=jax.ShapeDtypeStruct((BATCH, DPAD), jnp.float32),
        mesh=mesh,
        scratch_types=[
            pltpu.VMEM((NFEAT, bpb), jnp.int32),
            pltpu.VMEM((NFEAT, bpb + 16), jnp.int32),
            pltpu.VMEM((NFEAT, bcs, 128), jnp.float32),
            pltpu.VMEM((bcs, DPAD), jnp.float32),
            pltpu.SemaphoreType.DMA,
            pltpu.SemaphoreType.DMA,
        ],
    )
    def gather_kernel(tab_hbm, x_hbm, out_hbm, rows_v, offs_v, buf_v, slab_v,
                      gsem, wsem):
        wid = lax.axis_index("s") * NC + lax.axis_index("c")
        base = wid * bpb
        pltpu.sync_copy(x_hbm.at[:, pl.ds(base, bpb)], rows_v)

        def to_idx(k, _):
            f = k // (bpb // 16)
            l0 = lax.rem(k, bpb // 16) * 16
            g = rows_v[f, pl.ds(l0, 16)]
            row = ((g >> 11) << 9) + (g & 511)
            off = ((g >> 9) & 3) << 5
            offs_v[f, pl.ds(l0, 16)] = off
            rows_v[f, pl.ds(l0, 16)] = row
            return ()

        lax.fori_loop(0, NFEAT * (bpb // 16), to_idx, (), unroll=4)

        def per_chunk(bc, _):
            def fire(f, _):
                pltpu.async_copy(
                    tab_hbm.at[rows_v.at[f, pl.ds(bc * bcs, bcs)]],
                    buf_v.at[f],
                    gsem,
                )
                return ()

            lax.fori_loop(0, NFEAT, fire, (), unroll=False)

            def drain(f, _):
                pltpu.make_async_copy(
                    tab_hbm.at[rows_v.at[f, pl.ds(bc * bcs, bcs)]],
                    buf_v.at[f],
                    gsem,
                ).wait()
                return ()

            lax.fori_loop(0, NFEAT, drain, (), unroll=False)

            for f in range(NFEAT):
                ov = offs_v[f, pl.ds(bc * bcs, bcs)]
                for bl in range(bcs):
                    off = (ov[bl] >> 5) << 5
                    slab_v[bl, pl.ds(f * EMBED, 16)] = buf_v[
                        f, bl, pl.ds(off, 16)
                    ]
                    slab_v[bl, pl.ds(f * EMBED + 16, 16)] = buf_v[
                        f, bl, pl.ds(off + 16, 16)
                    ]

            pltpu.async_copy(
                slab_v, out_hbm.at[pl.ds(base + bc * bcs, bcs)], wsem
            )
            pltpu.make_async_copy(
                slab_v, out_hbm.at[pl.ds(base + bc * bcs, bcs)], wsem
            ).wait()
            return ()

        lax.fori_loop(0, bcn, per_chunk, (), unroll=False)

    return gather_kernel(table, x)


def _mlp_block(x_ref, w1_ref, b1_ref, w2_ref, b2_ref, o_ref):
    h = jnp.dot(x_ref[...], w1_ref[...], preferred_element_type=jnp.float32)
    h = jnp.maximum(h + b1_ref[...], 0.0)
    o_ref[...] = (
        jnp.dot(h, w2_ref[...], preferred_element_type=jnp.float32) + b2_ref[...]
    )


def _mlp(e, W1p, b1, W2, b2):
    bb = 512
    grid = (BATCH // bb,)
    return pl.pallas_call(
        _mlp_block,
        grid=grid,
        in_specs=[
            pl.BlockSpec((bb, DPAD), lambda i: (i, 0)),
            pl.BlockSpec((DPAD, HIDDEN), lambda i: (0, 0)),
            pl.BlockSpec((1, HIDDEN), lambda i: (0, 0)),
            pl.BlockSpec((HIDDEN, NCLASS), lambda i: (0, 0)),
            pl.BlockSpec((1, NCLASS), lambda i: (0, 0)),
        ],
        out_specs=pl.BlockSpec((bb, NCLASS), lambda i: (i, 0)),
        out_shape=jax.ShapeDtypeStruct((BATCH, NCLASS), jnp.float32),
    )(e, W1p, b1.reshape(1, HIDDEN), W2, b2.reshape(1, NCLASS))


@jax.jit
def kernel(x, emb, W1, b1, W2, b2):
    table = _pack_table(emb.T)
    e = _gather_rows(table, x)
    W1p = jnp.pad(W1, ((0, 64), (0, 0)))
    return _mlp(e, W1p, b1, W2, b2)


# speedup vs baseline: 7.7719x; 1.4607x over previous
import functools

import jax
import jax.numpy as jnp
from jax import lax
from jax.experimental import pallas as pl
from jax.experimental.pallas import tpu as pltpu
from jax.experimental.pallas import tpu_sc as plsc

VOCAB = 1000000
EMBED = 32
NFEAT = 26
BATCH = 4096
HIDDEN = 512
NCLASS = 2

NC = 2
NS = 16
NW = NC * NS

BC = 8192
NB = (VOCAB + BC - 1) // BC
TROWS = NB * (BC // 4)
DPAD = NFEAT * EMBED + 64


def _pack_table_block(x_ref, o_ref):
    t = jnp.transpose(x_ref[...], (1, 0))
    q = BC // 4
    o_ref[...] = jnp.concatenate(
        [t[s * q:(s + 1) * q, :] for s in range(4)], axis=1
    )


def _pack_table(embT):
    return pl.pallas_call(
        _pack_table_block,
        grid=(NB,),
        in_specs=[pl.BlockSpec((EMBED, BC), lambda i: (0, i))],
        out_specs=pl.BlockSpec((BC // 4, 128), lambda i: (i, 0)),
        out_shape=jax.ShapeDtypeStruct((TROWS, 128), jnp.float32),
    )(embT)


def _gather_rows(table, x):
    bpb = BATCH // NW
    bcn = 8
    bcs = bpb // bcn
    mesh = plsc.VectorSubcoreMesh(core_axis_name="c", subcore_axis_name="s")

    @functools.partial(
        pl.kernel,
        out_type=jax.ShapeDtypeStruct((BATCH, DPAD), jnp.float32),
        mesh=mesh,
        scratch_types=[
            pltpu.VMEM((NFEAT, bpb), jnp.int32),
            pltpu.VMEM((NFEAT, bpb + 16), jnp.int32),
            pltpu.VMEM((NFEAT, bcs, 128), jnp.float32),
            pltpu.VMEM((bcs, DPAD), jnp.float32),
            pltpu.SemaphoreType.DMA,
            pltpu.SemaphoreType.DMA,
        ],
    )
    def gather_kernel(tab_hbm, x_hbm, out_hbm, rows_v, offs_v, buf_v, slab_v,
                      gsem, wsem):
        wid = lax.axis_index("s") * NC + lax.axis_index("c")
        base = wid * bpb
        pltpu.sync_copy(x_hbm.at[:, pl.ds(base, bpb)], rows_v)

        def to_idx(k, _):
            f = k // (bpb // 16)
            l0 = lax.rem(k, bpb // 16) * 16
            g = rows_v[f, pl.ds(l0, 16)]
            row = ((g >> 13) << 11) + (g & 2047)
            off = ((g >> 11) & 3) << 5
            offs_v[f, pl.ds(l0, 16)] = off
            rows_v[f, pl.ds(l0, 16)] = row
            return ()

        lax.fori_loop(0, NFEAT * (bpb // 16), to_idx, (), unroll=4)

        def per_chunk(bc, _):
            def fire(f, _):
                pltpu.async_copy(
                    tab_hbm.at[rows_v.at[f, pl.ds(bc * bcs, bcs)]],
                    buf_v.at[f],
                    gsem,
                )
                return ()

            lax.fori_loop(0, NFEAT, fire, (), unroll=False)

            def drain(f, _):
                pltpu.make_async_copy(
                    tab_hbm.at[rows_v.at[f, pl.ds(bc * bcs, bcs)]],
                    buf_v.at[f],
                    gsem,
                ).wait()
                return ()

            lax.fori_loop(0, NFEAT, drain, (), unroll=False)

            for f in range(NFEAT):
                ov = offs_v[f, pl.ds(bc * bcs, bcs)]
                for bl in range(bcs):
                    off = (ov[bl] >> 5) << 5
                    slab_v[bl, pl.ds(f * EMBED, 16)] = buf_v[
                        f, bl, pl.ds(off, 16)
                    ]
                    slab_v[bl, pl.ds(f * EMBED + 16, 16)] = buf_v[
                        f, bl, pl.ds(off + 16, 16)
                    ]

            pltpu.async_copy(
                slab_v, out_hbm.at[pl.ds(base + bc * bcs, bcs)], wsem
            )
            pltpu.make_async_copy(
                slab_v, out_hbm.at[pl.ds(base + bc * bcs, bcs)], wsem
            ).wait()
            return ()

        lax.fori_loop(0, bcn, per_chunk, (), unroll=False)

    return gather_kernel(table, x)


def _mlp_block(x_ref, w1_ref, b1_ref, w2_ref, b2_ref, o_ref):
    h = jnp.dot(x_ref[...], w1_ref[...], preferred_element_type=jnp.float32)
    h = jnp.maximum(h + b1_ref[...], 0.0)
    o_ref[...] = (
        jnp.dot(h, w2_ref[...], preferred_element_type=jnp.float32) + b2_ref[...]
    )


def _mlp(e, W1p, b1, W2, b2):
    bb = 512
    grid = (BATCH // bb,)
    return pl.pallas_call(
        _mlp_block,
        grid=grid,
        in_specs=[
            pl.BlockSpec((bb, DPAD), lambda i: (i, 0)),
            pl.BlockSpec((DPAD, HIDDEN), lambda i: (0, 0)),
            pl.BlockSpec((1, HIDDEN), lambda i: (0, 0)),
            pl.BlockSpec((HIDDEN, NCLASS), lambda i: (0, 0)),
            pl.BlockSpec((1, NCLASS), lambda i: (0, 0)),
        ],
        out_specs=pl.BlockSpec((bb, NCLASS), lambda i: (i, 0)),
        out_shape=jax.ShapeDtypeStruct((BATCH, NCLASS), jnp.float32),
    )(e, W1p, b1.reshape(1, HIDDEN), W2, b2.reshape(1, NCLASS))


@jax.jit
def kernel(x, emb, W1, b1, W2, b2):
    table = _pack_table(emb.T)
    e = _gather_rows(table, x)
    W1p = jnp.pad(W1, ((0, 64), (0, 0)))
    return _mlp(e, W1p, b1, W2, b2)


# speedup vs baseline: 7.8713x; 1.0128x over previous
import functools

import jax
import jax.numpy as jnp
from jax import lax
from jax.experimental import pallas as pl
from jax.experimental.pallas import tpu as pltpu
from jax.experimental.pallas import tpu_sc as plsc

VOCAB = 1000000
EMBED = 32
NFEAT = 26
BATCH = 4096
HIDDEN = 512
NCLASS = 2

NC = 2
NS = 16
NW = NC * NS

BC = 16384
NB = (VOCAB + BC - 1) // BC
TROWS = NB * (BC // 4)
DPAD = NFEAT * EMBED + 64


def _pack_table_block(x_ref, o_ref):
    t = jnp.transpose(x_ref[...], (1, 0))
    q = BC // 4
    o_ref[...] = jnp.concatenate(
        [t[s * q:(s + 1) * q, :] for s in range(4)], axis=1
    )


def _pack_table(embT):
    return pl.pallas_call(
        _pack_table_block,
        grid=(NB,),
        in_specs=[pl.BlockSpec((EMBED, BC), lambda i: (0, i))],
        out_specs=pl.BlockSpec((BC // 4, 128), lambda i: (i, 0)),
        out_shape=jax.ShapeDtypeStruct((TROWS, 128), jnp.float32),
    )(embT)


def _gather_rows(table, x):
    bpb = BATCH // NW
    bcn = 8
    bcs = bpb // bcn
    mesh = plsc.VectorSubcoreMesh(core_axis_name="c", subcore_axis_name="s")

    @functools.partial(
        pl.kernel,
        out_type=jax.ShapeDtypeStruct((BATCH, DPAD), jnp.float32),
        mesh=mesh,
        scratch_types=[
            pltpu.VMEM((NFEAT, bpb), jnp.int32),
            pltpu.VMEM((NFEAT, bpb + 16), jnp.int32),
            pltpu.VMEM((NFEAT, bcs, 128), jnp.float32),
            pltpu.VMEM((bcs, DPAD), jnp.float32),
            pltpu.SemaphoreType.DMA,
            pltpu.SemaphoreType.DMA,
        ],
    )
    def gather_kernel(tab_hbm, x_hbm, out_hbm, rows_v, offs_v, buf_v, slab_v,
                      gsem, wsem):
        wid = lax.axis_index("s") * NC + lax.axis_index("c")
        base = wid * bpb
        pltpu.sync_copy(x_hbm.at[:, pl.ds(base, bpb)], rows_v)

        def to_idx(k, _):
            f = k // (bpb // 16)
            l0 = lax.rem(k, bpb // 16) * 16
            g = rows_v[f, pl.ds(l0, 16)]
            row = ((g >> 14) << 12) + (g & 4095)
            off = ((g >> 12) & 3) << 5
            offs_v[f, pl.ds(l0, 16)] = off
            rows_v[f, pl.ds(l0, 16)] = row
            return ()

        lax.fori_loop(0, NFEAT * (bpb // 16), to_idx, (), unroll=4)

        def per_chunk(bc, _):
            def fire(f, _):
                pltpu.async_copy(
                    tab_hbm.at[rows_v.at[f, pl.ds(bc * bcs, bcs)]],
                    buf_v.at[f],
                    gsem,
                )
                return ()

            lax.fori_loop(0, NFEAT, fire, (), unroll=False)

            def drain(f, _):
                pltpu.make_async_copy(
                    tab_hbm.at[rows_v.at[f, pl.ds(bc * bcs, bcs)]],
                    buf_v.at[f],
                    gsem,
                ).wait()
                return ()

            lax.fori_loop(0, NFEAT, drain, (), unroll=False)

            for f in range(NFEAT):
                ov = offs_v[f, pl.ds(bc * bcs, bcs)]
                for bl in range(bcs):
                    off = (ov[bl] >> 5) << 5
                    slab_v[bl, pl.ds(f * EMBED, 16)] = buf_v[
                        f, bl, pl.ds(off, 16)
                    ]
                    slab_v[bl, pl.ds(f * EMBED + 16, 16)] = buf_v[
                        f, bl, pl.ds(off + 16, 16)
                    ]

            pltpu.async_copy(
                slab_v, out_hbm.at[pl.ds(base + bc * bcs, bcs)], wsem
            )
            pltpu.make_async_copy(
                slab_v, out_hbm.at[pl.ds(base + bc * bcs, bcs)], wsem
            ).wait()
            return ()

        lax.fori_loop(0, bcn, per_chunk, (), unroll=False)

    return gather_kernel(table, x)


def _mlp_block(x_ref, w1_ref, b1_ref, w2_ref, b2_ref, o_ref):
    h = jnp.dot(x_ref[...], w1_ref[...], preferred_element_type=jnp.float32)
    h = jnp.maximum(h + b1_ref[...], 0.0)
    o_ref[...] = (
        jnp.dot(h, w2_ref[...], preferred_element_type=jnp.float32) + b2_ref[...]
    )


def _mlp(e, W1p, b1, W2, b2):
    bb = 512
    grid = (BATCH // bb,)
    return pl.pallas_call(
        _mlp_block,
        grid=grid,
        in_specs=[
            pl.BlockSpec((bb, DPAD), lambda i: (i, 0)),
            pl.BlockSpec((DPAD, HIDDEN), lambda i: (0, 0)),
            pl.BlockSpec((1, HIDDEN), lambda i: (0, 0)),
            pl.BlockSpec((HIDDEN, NCLASS), lambda i: (0, 0)),
            pl.BlockSpec((1, NCLASS), lambda i: (0, 0)),
        ],
        out_specs=pl.BlockSpec((bb, NCLASS), lambda i: (i, 0)),
        out_shape=jax.ShapeDtypeStruct((BATCH, NCLASS), jnp.float32),
    )(e, W1p, b1.reshape(1, HIDDEN), W2, b2.reshape(1, NCLASS))


@jax.jit
def kernel(x, emb, W1, b1, W2, b2):
    table = _pack_table(emb.T)
    e = _gather_rows(table, x)
    W1p = jnp.pad(W1, ((0, 64), (0, 0)))
    return _mlp(e, W1p, b1, W2, b2)
